# baseline (speedup 1.0000x reference)
import functools

import jax
import jax.numpy as jnp
from jax.experimental import pallas as pl
from jax.experimental.pallas import tpu as pltpu

_N = 20000
_B = 512
_ROWS = 2048


def _iou_kernel(b1_ref, b2t_ref, ious_ref, amax_ref):
    b1 = b1_ref[...]
    b2 = b2t_ref[...]

    im_a = b1[:, 0:1]
    x1a = b1[:, 1:2]
    y1a = b1[:, 2:3]
    x2a = b1[:, 3:4]
    y2a = b1[:, 4:5]
    area_a = b1[:, 5:6]

    im_b = b2[0:1, :]
    x1b = b2[1:2, :]
    y1b = b2[2:3, :]
    x2b = b2[3:4, :]
    y2b = b2[4:5, :]
    area_b = b2[5:6, :]

    iw = jnp.maximum(jnp.minimum(x2a, x2b) - jnp.maximum(x1a, x1b), 0.0)
    ih = jnp.maximum(jnp.minimum(y2a, y2b) - jnp.maximum(y1a, y1b), 0.0)
    inter = iw * ih
    iou = inter / ((area_a + area_b) - inter)
    iou = jnp.where(im_a != im_b, 0.0, iou)
    ious_ref[...] = iou

    mx = jnp.max(iou, axis=1, keepdims=True)
    col = jax.lax.broadcasted_iota(jnp.int32, iou.shape, 1)
    amax_ref[...] = jnp.min(
        jnp.where(iou == mx, col, _B), axis=1, keepdims=True
    )


def _pack(boxes):
    im = boxes[:, 0:1]
    x1 = boxes[:, 1:2]
    y1 = boxes[:, 2:3]
    x2 = boxes[:, 3:4]
    y2 = boxes[:, 4:5]
    area = (x2 - x1 + 1.0) * (y2 - y1 + 1.0)
    return jnp.concatenate([im, x1, y1, x2 + 1.0, y2 + 1.0, area], axis=1)


@functools.partial(jax.jit, static_argnames=())
def kernel(boxes1, boxes2):
    b1p = _pack(boxes1)
    b2p = _pack(boxes2).T
    grid = (pl.cdiv(_N, _ROWS),)
    ious, amax = pl.pallas_call(
        _iou_kernel,
        grid=grid,
        in_specs=[
            pl.BlockSpec((_ROWS, 6), lambda i: (i, 0)),
            pl.BlockSpec((6, _B), lambda i: (0, 0)),
        ],
        out_specs=[
            pl.BlockSpec((_ROWS, _B), lambda i: (i, 0)),
            pl.BlockSpec((_ROWS, 1), lambda i: (i, 0)),
        ],
        out_shape=[
            jax.ShapeDtypeStruct((_N, _B), jnp.float32),
            jax.ShapeDtypeStruct((_N, 1), jnp.int32),
        ],
        compiler_params=pltpu.CompilerParams(
            dimension_semantics=("parallel",),
        ),
    )(b1p, b2p)
    return amax.reshape(_N), ious
